# baseline (speedup 1.0000x reference)
import jax
import jax.numpy as jnp
from jax.experimental import pallas as pl
from jax.experimental.pallas import tpu as pltpu

TOPK = 64
R_BLK = 128
C_BLK = 2048
CHUNK = 2048

INT_MIN = -2147483648
NEG_FINITE_MIN = INT_MIN + 0x800000

BISECT_STEPS = 6


def _decode(key):
    bits = jnp.where(key < 0, key ^ jnp.int32(0x7FFFFFFF), key)
    return jax.lax.bitcast_convert_type(bits, jnp.float32)


def _body(x_ref, w_ref, b_ref, o_ref, slab_ref, t_ref):
    i = pl.program_id(0)
    j = pl.program_id(1)
    ni = pl.num_programs(0) - 1
    nj = pl.num_programs(1)
    cur = jax.lax.rem(i, 2)
    prev = jax.lax.rem(i + 1, 2)
    d_sae = o_ref.shape[1]
    rows = o_ref.shape[0]

    @pl.when(i < ni)
    def _matmul():
        acc = jax.lax.dot_general(
            x_ref[...], w_ref[...],
            dimension_numbers=(((1,), (1,)), ((), ())),
            preferred_element_type=jnp.float32,
        )
        slab_ref[cur, :, pl.ds(j * C_BLK, C_BLK)] = (
            acc + b_ref[0, pl.ds(j * C_BLK, C_BLK)][None, :])

    @pl.when(i >= 1)
    def _select():
        n_chunks = d_sae // CHUNK

        @pl.when(j == 0)
        def _init():
            t_ref[...] = jnp.full((rows, 1), INT_MIN, jnp.int32)

        @pl.when(j < BISECT_STEPS)
        def _bisect():
            def count_ge(candf):
                def cbody(c, acc):
                    blk = slab_ref[prev, :, pl.ds(c * CHUNK, CHUNK)]
                    return acc + jnp.sum((blk >= candf).astype(jnp.int32),
                                         axis=1, keepdims=True)
                return jax.lax.fori_loop(0, n_chunks, cbody,
                                         jnp.zeros((rows, 1), jnp.int32))

            def bbody(it, t):
                shift = (jnp.int32(31) - it).astype(jnp.uint32)
                cand = t + (jnp.int32(1) << shift)
                cnt = count_ge(_decode(cand))
                cnt = jnp.where(cand < jnp.int32(NEG_FINITE_MIN),
                                jnp.int32(d_sae), cnt)
                return jnp.where(cnt >= TOPK, cand, t)

            start = 5 * j + jnp.minimum(j, 2)
            niter = jnp.where(j < 2, 6, 5)
            t_ref[...] = jax.lax.fori_loop(start, start + niter, bbody,
                                           t_ref[...])

        @pl.when(j >= BISECT_STEPS)
        def _mask_write():
            tf = _decode(t_ref[...])
            half = d_sae // (nj - BISECT_STEPS)
            base = (j - BISECT_STEPS) * half

            def mbody(c, _):
                sl = pl.ds(base + c * CHUNK, CHUNK)
                blk = slab_ref[prev, :, sl]
                o_ref[:, sl] = jnp.where(blk >= tf, blk, 0.0)
                return 0
            jax.lax.fori_loop(0, half // CHUNK, mbody, 0)


def kernel(x, W_enc, b_enc):
    n_tok, d_model = x.shape
    d_sae = W_enc.shape[0]
    b2 = b_enc.reshape(1, d_sae)
    xb = x.astype(jnp.bfloat16)
    wb = W_enc.astype(jnp.bfloat16)
    ni = n_tok // R_BLK
    nj = d_sae // C_BLK
    return pl.pallas_call(
        _body,
        grid=(ni + 1, nj),
        in_specs=[
            pl.BlockSpec((R_BLK, d_model), lambda i, j: (jnp.minimum(i, ni - 1), 0)),
            pl.BlockSpec((C_BLK, d_model), lambda i, j: (j, 0)),
            pl.BlockSpec((1, d_sae), lambda i, j: (0, 0)),
        ],
        out_specs=pl.BlockSpec((R_BLK, d_sae), lambda i, j: (jnp.maximum(i - 1, 0), 0)),
        out_shape=jax.ShapeDtypeStruct((n_tok, d_sae), jnp.float32),
        scratch_shapes=[
            pltpu.VMEM((2, R_BLK, d_sae), jnp.float32),
            pltpu.VMEM((R_BLK, 1), jnp.int32),
        ],
        compiler_params=pltpu.CompilerParams(
            dimension_semantics=("arbitrary", "arbitrary"),
        ),
    )(xb, wb, b2)


# speedup vs baseline: 1.4460x; 1.4460x over previous
import jax
import jax.numpy as jnp
from jax.experimental import pallas as pl
from jax.experimental.pallas import tpu as pltpu

TOPK = 64
R_BLK = 256
C_BLK = 1024
CHUNK = 2048

INT_MIN = -2147483648
NEG_FINITE_MIN = INT_MIN + 0x800000

BISECT_STEPS = 12
OUT_SPLITS = 4


def _decode(key):
    bits = jnp.where(key < 0, key ^ jnp.int32(0x7FFFFFFF), key)
    return jax.lax.bitcast_convert_type(bits, jnp.float32)


def _body(x_ref, w_ref, b_ref, o_ref, slab_ref, t_ref):
    i = pl.program_id(0)
    j = pl.program_id(1)
    ni = pl.num_programs(0) - 1
    cur = jax.lax.rem(i, 2)
    prev = jax.lax.rem(i + 1, 2)
    rows = R_BLK
    d_sae = slab_ref.shape[2]

    @pl.when(i < ni)
    def _matmul():
        acc = jax.lax.dot_general(
            x_ref[...], w_ref[...],
            dimension_numbers=(((1,), (1,)), ((), ())),
            preferred_element_type=jnp.float32,
        )
        slab_ref[cur, :, pl.ds(j * C_BLK, C_BLK)] = (
            acc + b_ref[0, pl.ds(j * C_BLK, C_BLK)][None, :])

    @pl.when(i >= 1)
    def _select():
        n_chunks = d_sae // CHUNK

        @pl.when(j == 0)
        def _init():
            t_ref[...] = jnp.full((rows, 1), INT_MIN, jnp.int32)

        @pl.when(j < BISECT_STEPS)
        def _bisect():
            def count_ge(candf):
                def cbody(c, acc):
                    blk = slab_ref[prev, :, pl.ds(c * CHUNK, CHUNK)]
                    return acc + jnp.sum((blk >= candf).astype(jnp.int32),
                                         axis=1, keepdims=True)
                return jax.lax.fori_loop(0, n_chunks, cbody,
                                         jnp.zeros((rows, 1), jnp.int32))

            def bbody(it, t):
                shift = (jnp.int32(31) - it).astype(jnp.uint32)
                cand = t + (jnp.int32(1) << shift)
                cnt = count_ge(_decode(cand))
                cnt = jnp.where(cand < jnp.int32(NEG_FINITE_MIN),
                                jnp.int32(d_sae), cnt)
                return jnp.where(cnt >= TOPK, cand, t)

            start = 3 * jnp.minimum(j, 8) + 2 * jnp.maximum(j - 8, 0)
            niter = jnp.where(j < 8, 3, 2)
            t_ref[...] = jax.lax.fori_loop(start, start + niter, bbody,
                                           t_ref[...])

        @pl.when(j >= BISECT_STEPS)
        def _mask_write():
            tf = _decode(t_ref[...])
            quarter = d_sae // OUT_SPLITS
            base = (j - BISECT_STEPS) * quarter

            def mbody(c, _):
                sl = pl.ds(base + c * CHUNK, CHUNK)
                blk = slab_ref[prev, :, sl]
                o_ref[:, pl.ds(c * CHUNK, CHUNK)] = jnp.where(blk >= tf, blk, 0.0)
                return 0
            jax.lax.fori_loop(0, quarter // CHUNK, mbody, 0)


def kernel(x, W_enc, b_enc):
    n_tok, d_model = x.shape
    d_sae = W_enc.shape[0]
    b2 = b_enc.reshape(1, d_sae)
    xb = x.astype(jnp.bfloat16)
    wb = W_enc.astype(jnp.bfloat16)
    ni = n_tok // R_BLK
    nj = d_sae // C_BLK

    def out_idx(i, j):
        ii = jnp.maximum(i - 1, 0)
        jj = jnp.where(i == 0, 0, jnp.clip(j - BISECT_STEPS, 0, OUT_SPLITS - 1))
        return ii, jj

    return pl.pallas_call(
        _body,
        grid=(ni + 1, nj),
        in_specs=[
            pl.BlockSpec((R_BLK, d_model), lambda i, j: (jnp.minimum(i, ni - 1), 0)),
            pl.BlockSpec((C_BLK, d_model), lambda i, j: (j, 0)),
            pl.BlockSpec((1, d_sae), lambda i, j: (0, 0)),
        ],
        out_specs=pl.BlockSpec((R_BLK, d_sae // OUT_SPLITS), out_idx),
        out_shape=jax.ShapeDtypeStruct((n_tok, d_sae), jnp.float32),
        scratch_shapes=[
            pltpu.VMEM((2, R_BLK, d_sae), jnp.float32),
            pltpu.VMEM((R_BLK, 1), jnp.int32),
        ],
        compiler_params=pltpu.CompilerParams(
            dimension_semantics=("arbitrary", "arbitrary"),
        ),
    )(xb, wb, b2)
